# baseline (speedup 1.0000x reference)
import functools

import jax
import jax.numpy as jnp
from jax import lax
from jax.experimental import pallas as pl
from jax.experimental.pallas import tpu as pltpu
from jax.experimental.pallas import tpu_sc as plsc

N1 = 10000
N2 = 6000
C = 128
H = 64

NC = 2
NS = 16
NW = NC * NS
CH = 128

NP = 10240
RPT = NP // NS

NCH_S = 160
NCH_C = 79


def _ceil_to(x, m):
    return (x + m - 1) // m * m


def _make_s_kernel():
    mesh = plsc.VectorSubcoreMesh(core_axis_name="c", subcore_axis_name="s")

    @functools.partial(
        pl.kernel,
        out_type=jax.ShapeDtypeStruct((NP, NC, H), jnp.float32),
        mesh=mesh,
        compiler_params=pltpu.CompilerParams(use_tc_tiling_on_sc=False),
        scratch_types=[
            pltpu.VMEM((16,), jnp.int32),
            pltpu.VMEM((NCH_S, CH), jnp.int32),
            pltpu.VMEM((NCH_S, CH), jnp.int32),
            pltpu.VMEM((1, CH, H), jnp.float32),
            pltpu.VMEM((1, CH, H), jnp.float32),
            pltpu.VMEM((CH, H), jnp.float32),
            pltpu.VMEM_SHARED((NP, H), jnp.float32),
            pltpu.SemaphoreType.DMA,
            pltpu.SemaphoreType.DMA,
            pltpu.SemaphoreType.DMA,
            pltpu.SemaphoreType.DMA,
        ],
    )
    def s_kernel(g_hbm, src_hbm, dst_hbm, nch_hbm, out_hbm,
                 nch_v, src_v, dst_v, bufa_v, bufb_v, zrow_v, acc_sh,
                 sga, ssa, sgb, ssb):
        cid = lax.axis_index("c")
        sid = lax.axis_index("s")

        pltpu.sync_copy(nch_hbm, nch_v)
        nv = nch_v[...]
        n_live = nv[0]
        nz = nv[1]
        row0 = sid * nz * CH

        def _zfill(i, _):
            for k in range(H // 16):
                zrow_v[i, pl.ds(k * 16, 16)] = jnp.zeros((16,), jnp.float32)
            return 0
        lax.fori_loop(0, CH, _zfill, 0)

        def _zero(z, _):
            pltpu.sync_copy(zrow_v, acc_sh.at[pl.ds(row0 + z * CH, CH)])
            return 0
        lax.fori_loop(0, nz, _zero, 0)
        plsc.subcore_barrier()

        pltpu.sync_copy(src_hbm.at[sid], src_v)
        pltpu.sync_copy(dst_hbm.at[sid], dst_v)

        rows_v = bufa_v.at[0]

        def _chunk(j, _):
            pltpu.async_copy(
                g_hbm.at[cid].at[src_v.at[j]], rows_v, sga).wait()
            pltpu.sync_copy(rows_v, acc_sh.at[dst_v.at[j]], add=True)
            return 0
        lax.fori_loop(0, n_live, _chunk, 0)

        plsc.subcore_barrier()

        def _dump(z, _):
            pltpu.sync_copy(acc_sh.at[pl.ds(row0 + z * CH, CH)],
                            out_hbm.at[pl.ds(row0 + z * CH, CH), cid])
            return 0
        lax.fori_loop(0, nz, _dump, 0)

    return s_kernel


def _make_count_kernel():
    W = 16
    mesh = plsc.VectorSubcoreMesh(core_axis_name="c", subcore_axis_name="s")

    @functools.partial(
        pl.kernel,
        out_type=jax.ShapeDtypeStruct((3, NC, NP, W), jnp.float32),
        mesh=mesh,
        compiler_params=pltpu.CompilerParams(use_tc_tiling_on_sc=False),
        scratch_types=[
            pltpu.VMEM((16,), jnp.int32),
            pltpu.VMEM((NCH_C, CH), jnp.int32),
            pltpu.VMEM((CH, W), jnp.float32),
            pltpu.VMEM((CH, W), jnp.float32),
            pltpu.VMEM_SHARED((NP, W), jnp.float32),
        ],
    )
    def count_kernel(dst_hbm, nch_hbm, out_hbm,
                     nch_v, dst_v, ones_v, zrow_v, acc_sh):
        cid = lax.axis_index("c")
        sid = lax.axis_index("s")
        wid = cid * NS + sid

        def _fill(i, _):
            zrow_v[i, pl.ds(0, 16)] = jnp.zeros((16,), jnp.float32)
            ones_v[i, pl.ds(0, 16)] = jnp.ones((16,), jnp.float32)
            return 0
        lax.fori_loop(0, CH, _fill, 0)
        pltpu.sync_copy(nch_hbm, nch_v)
        nv = nch_v[...]

        for p in range(3):
            n_live = nv[p]
            nz = nv[3 + p]
            row0 = sid * nz * CH

            def _zero(z, _):
                pltpu.sync_copy(zrow_v, acc_sh.at[pl.ds(row0 + z * CH, CH)])
                return 0
            lax.fori_loop(0, nz, _zero, 0)
            plsc.subcore_barrier()

            pltpu.sync_copy(dst_hbm.at[wid, p], dst_v)

            def _chunk(j, _):
                pltpu.sync_copy(ones_v, acc_sh.at[dst_v.at[j]], add=True)
                return 0
            lax.fori_loop(0, n_live, _chunk, 0)

            plsc.subcore_barrier()

            def _dump(z, _):
                pltpu.sync_copy(acc_sh.at[pl.ds(row0 + z * CH, CH)],
                                out_hbm.at[p, cid, pl.ds(row0 + z * CH, CH)])
                return 0
            lax.fori_loop(0, nz, _dump, 0)

    return count_kernel


def _lrelu(h):
    return jnp.where(h >= 0, h, 0.01 * h)


def _split_halves(g, out_ref):
    out_ref[0] = g[:, :H]
    out_ref[1] = g[:, H:]


def _make_dis_kernel():
    def body(cnt_ref, dis1_ref, dis2_ref):
        row = lax.broadcasted_iota(jnp.int32, (NP, 1), 0)
        for p, n_real, out_ref in ((0, N1, dis1_ref), (1, N2, dis2_ref)):
            d = cnt_ref[p, 0, :, 0:1] + cnt_ref[p, 1, :, 0:1]
            dis = jnp.where(d > 0, lax.rsqrt(jnp.maximum(d, 1e-30)), 0.0)
            dis = jnp.where(row < n_real, dis, 0.0)
            out_ref[...] = jnp.broadcast_to(dis, (NP, C))

    return pl.pallas_call(
        body, out_shape=(jax.ShapeDtypeStruct((NP, C), jnp.float32),
                         jax.ShapeDtypeStruct((NP, C), jnp.float32)))


def _make_scale_kernel():
    def body(h_ref, dis_ref, out_ref):
        _split_halves(h_ref[...] * dis_ref[...], out_ref)

    return pl.pallas_call(
        body, out_shape=jax.ShapeDtypeStruct((NC, NP, H), jnp.float32))


def _make_scale2_kernel():
    def body(s_ref, dis_ref, out_ref):
        d = dis_ref[...]
        _split_halves(-(d * d) * s_ref[...], out_ref)

    return pl.pallas_call(
        body, out_shape=jax.ShapeDtypeStruct((NC, NP, H), jnp.float32))


@functools.lru_cache(maxsize=None)
def _make_layer_kernel(n_real, do_bn, scale_next):
    inv_n = 1.0 / n_real

    def body(tx0_ref, s1_ref, s2_ref, dis_ref, w_ref, b_ref, gam_ref,
             bet_ref, disn_ref, y_ref, gy_ref):
        tx0 = tx0_ref[...]
        d = dis_ref[...]
        tx1 = -d * s1_ref[...]
        tx2 = -2.0 * d * s2_ref[...] - tx0
        pre = (jnp.dot(tx0, w_ref[0], preferred_element_type=jnp.float32)
               + jnp.dot(tx1, w_ref[1], preferred_element_type=jnp.float32)
               + jnp.dot(tx2, w_ref[2], preferred_element_type=jnp.float32)
               + b_ref[...])
        row = lax.broadcasted_iota(jnp.int32, (NP, C), 0)
        pre = jnp.where(row < n_real, pre, 0.0)
        if do_bn:
            m = jnp.sum(pre, axis=0, keepdims=True) * inv_n
            v = jnp.sum(pre * pre, axis=0, keepdims=True) * inv_n - m * m
            y = gam_ref[...] * (pre - m) * lax.rsqrt(v + 1e-5) + bet_ref[...]
            y = _lrelu(y)
        else:
            y = pre
        y = jnp.where(row < n_real, y, 0.0)
        y_ref[...] = y
        if scale_next:
            _split_halves(disn_ref[...] * y, gy_ref)
        else:
            _split_halves(y, gy_ref)

    out_shape = (jax.ShapeDtypeStruct((NP, C), jnp.float32),
                 jax.ShapeDtypeStruct((NC, NP, H), jnp.float32))
    return pl.pallas_call(body, out_shape=out_shape)


@functools.lru_cache(maxsize=None)
def _make_pool_kernel(n_real):
    inv_n = 1.0 / n_real

    def body(ps_ref, cntp_ref, gam_ref, bet_ref, dis_ref, y_ref, gy_ref):
        cnt = cntp_ref[2, 0, :, 0:1] + cntp_ref[2, 1, :, 0:1]
        pooled = ps_ref[...] / jnp.broadcast_to(
            jnp.maximum(cnt, 1.0), (NP, C))
        row = lax.broadcasted_iota(jnp.int32, (NP, C), 0)
        pooled = jnp.where(row < n_real, pooled, 0.0)
        m = jnp.sum(pooled, axis=0, keepdims=True) * inv_n
        v = jnp.sum(pooled * pooled, axis=0, keepdims=True) * inv_n - m * m
        y = gam_ref[...] * (pooled - m) * lax.rsqrt(v + 1e-5) + bet_ref[...]
        y = _lrelu(y)
        row = lax.broadcasted_iota(jnp.int32, (NP, C), 0)
        y = jnp.where(row < n_real, y, 0.0)
        y_ref[...] = y
        _split_halves(dis_ref[...] * y, gy_ref)

    out_shape = (jax.ShapeDtypeStruct((NP, C), jnp.float32),
                 jax.ShapeDtypeStruct((NC, NP, H), jnp.float32))
    return pl.pallas_call(body, out_shape=out_shape)


def _pad_edges(idx, pad_val, ways, nch_max, even=False):
    e = idx.shape[0]
    per = ways * CH
    ep = _ceil_to(e, per)
    live = ep // per
    idx = jnp.concatenate([idx, jnp.full((ways * nch_max * CH - e,),
                                         pad_val, jnp.int32)])
    real = idx[:ep].reshape(live, ways, CH).transpose(1, 0, 2)
    fill = jnp.full((ways, nch_max - live, CH), pad_val, jnp.int32)
    return jnp.concatenate([real, fill], axis=1), live


def kernel(x, edge_index1, edge_index2, pool_idx,
           W1, b1, g1, be1, W2, b2, g2, be2,
           W3, b3, g3, be3, W4, b4, g4, be4, W5, b5, g5, be5):
    xp = jnp.zeros((NP, C), jnp.float32).at[:N1].set(x)

    src1, live1 = _pad_edges(edge_index1[0], N1, NS, NCH_S)
    dst1, _ = _pad_edges(edge_index1[1], N1, NS, NCH_S)
    src2, live2 = _pad_edges(edge_index2[0], N2, NS, NCH_S)
    dst2, _ = _pad_edges(edge_index2[1], N2, NS, NCH_S)
    psrc, livep = _pad_edges(jnp.arange(N1, dtype=jnp.int32), N1, NS, NCH_S)
    pdst, _ = _pad_edges(pool_idx.astype(jnp.int32), N2, NS, NCH_S)

    cdst1, clive1 = _pad_edges(edge_index1[1], N1, NW, NCH_C)
    cdst2, clive2 = _pad_edges(edge_index2[1], N2, NW, NCH_C)
    cpdst, clivep = _pad_edges(pool_idx.astype(jnp.int32), N2, NW, NCH_C)

    NZ1 = NP // (NS * CH)
    NZ2 = _ceil_to(N2, NS * CH) // (NS * CH)

    def nvec(live, nz):
        v = [live, nz] + [0] * 14
        return jnp.array(v, jnp.int32)

    sk = _make_s_kernel()
    ck = _make_count_kernel()
    scale = _make_scale_kernel()
    scale2 = _make_scale2_kernel()

    def S(g_halves, src, dst, live, nz):
        return sk(g_halves, src, dst, nvec(live, nz)).reshape(NP, C)

    cdst = jnp.stack([cdst1, cdst2, cpdst], axis=1)
    cnts = ck(cdst, jnp.array(
        [clive1, clive2, clivep, NZ1, NZ2, NZ2] + [0] * 10, jnp.int32))
    dis1, dis2 = _make_dis_kernel()(cnts)

    def cheb_hops(gh, src, dst, live, nz, dis):
        s1 = S(gh, src, dst, live, nz)
        s2 = S(scale2(s1, dis), src, dst, live, nz)
        return s1, s2

    s1, s2 = cheb_hops(scale(xp, dis1), src1, dst1, live1, NZ1, dis1)
    y1, gy1 = _make_layer_kernel(N1, True, True)(
        xp, s1, s2, dis1, W1, b1, g1, be1, dis1)

    s1, s2 = cheb_hops(gy1, src1, dst1, live1, NZ1, dis1)
    h2, h2_halves = _make_layer_kernel(N1, False, False)(
        y1, s1, s2, dis1, W2, b2, g2, be2, dis1)

    ps = S(h2_halves, psrc, pdst, livep, NZ2)
    y, gy = _make_pool_kernel(N2)(ps, cnts, g2, be2, dis2)

    for W, b, gam, bet in ((W3, b3, g3, be3),
                           (W4, b4, g4, be4),
                           (W5, b5, g5, be5)):
        s1, s2 = cheb_hops(gy, src2, dst2, live2, NZ2, dis2)
        y, gy = _make_layer_kernel(N2, True, True)(
            y, s1, s2, dis2, W, b, gam, bet, dis2)

    return y[:N2]


def _make_split_kernel():
    def body(h_ref, out_ref):
        _split_halves(h_ref[...], out_ref)

    return pl.pallas_call(
        body, out_shape=jax.ShapeDtypeStruct((NC, NP, H), jnp.float32))


def _split2(h):
    return _make_split_kernel()(h)


# speedup vs baseline: 1.0835x; 1.0835x over previous
import functools

import jax
import jax.numpy as jnp
from jax import lax
from jax.experimental import pallas as pl
from jax.experimental.pallas import tpu as pltpu
from jax.experimental.pallas import tpu_sc as plsc

N1 = 10000
N2 = 6000
C = 128
H = 64

NC = 2
NS = 16
NW = NC * NS
CH = 128

NP = 10240
RPT = NP // NS

NCH_S = 160
NCH_C = 79


def _ceil_to(x, m):
    return (x + m - 1) // m * m


def _make_s_kernel():
    mesh = plsc.VectorSubcoreMesh(core_axis_name="c", subcore_axis_name="s")

    @functools.partial(
        pl.kernel,
        out_type=(jax.ShapeDtypeStruct((NP, NC, H), jnp.float32),
                  jax.ShapeDtypeStruct((NC, NP, H), jnp.float32)),
        mesh=mesh,
        compiler_params=pltpu.CompilerParams(use_tc_tiling_on_sc=False),
        scratch_types=[
            pltpu.VMEM((16,), jnp.int32),
            pltpu.VMEM((NCH_S, CH), jnp.int32),
            pltpu.VMEM((NCH_S, CH), jnp.int32),
            pltpu.VMEM((1, CH, H), jnp.float32),
            pltpu.VMEM((1, CH, H), jnp.float32),
            pltpu.VMEM((CH, H), jnp.float32),
            pltpu.VMEM((RPT,), jnp.float32),
            pltpu.VMEM_SHARED((NP, H), jnp.float32),
            pltpu.SemaphoreType.DMA,
            pltpu.SemaphoreType.DMA,
            pltpu.SemaphoreType.DMA,
            pltpu.SemaphoreType.DMA,
        ],
    )
    def s_kernel(g_hbm, src_hbm, dst_hbm, nch_hbm, dq_hbm,
                 out_hbm, g2_hbm,
                 nch_v, src_v, dst_v, bufa_v, bufb_v, zrow_v, dq_v, acc_sh,
                 sga, ssa, sgb, ssb):
        cid = lax.axis_index("c")
        sid = lax.axis_index("s")

        pltpu.sync_copy(nch_hbm, nch_v)
        nv = nch_v[...]
        n_live = nv[0]
        nz = nv[1]
        row0 = sid * nz * CH

        def _zfill(i, _):
            for k in range(H // 16):
                zrow_v[i, pl.ds(k * 16, 16)] = jnp.zeros((16,), jnp.float32)
            return 0
        lax.fori_loop(0, CH, _zfill, 0)

        def _zero(z, _):
            pltpu.sync_copy(zrow_v, acc_sh.at[pl.ds(row0 + z * CH, CH)])
            return 0
        lax.fori_loop(0, nz, _zero, 0)
        plsc.subcore_barrier()

        pltpu.sync_copy(src_hbm.at[sid], src_v)
        pltpu.sync_copy(dst_hbm.at[sid], dst_v)

        rows_v = bufa_v.at[0]

        def _chunk(j, _):
            pltpu.async_copy(
                g_hbm.at[cid].at[src_v.at[j]], rows_v, sga).wait()
            pltpu.sync_copy(rows_v, acc_sh.at[dst_v.at[j]], add=True)
            return 0
        lax.fori_loop(0, n_live, _chunk, 0)

        plsc.subcore_barrier()

        def _dump(z, _):
            pltpu.sync_copy(acc_sh.at[pl.ds(row0 + z * CH, CH)],
                            out_hbm.at[pl.ds(row0 + z * CH, CH), cid])
            return 0
        lax.fori_loop(0, nz, _dump, 0)

        want_g2 = nv[2]

        @pl.when(want_g2 == 1)
        def _scaled_dump():
            pltpu.sync_copy(dq_hbm.at[pl.ds(row0, RPT)], dq_v)
            buf = bufa_v.at[0]
            sbuf = bufb_v.at[0]

            def _sdump(z, _):
                zoff = z * CH
                pltpu.sync_copy(acc_sh.at[pl.ds(row0 + zoff, CH)], buf)
                for g in range(CH // 16):
                    dq16 = dq_v[pl.ds(zoff + g * 16, 16)]
                    for k in range(16):
                        r = g * 16 + k
                        s = dq16[k]
                        for c in range(H // 16):
                            sbuf[r, pl.ds(c * 16, 16)] = (
                                buf[r, pl.ds(c * 16, 16)] * s)
                pltpu.sync_copy(
                    sbuf, g2_hbm.at[cid, pl.ds(row0 + zoff, CH)])
                return 0
            lax.fori_loop(0, nz, _sdump, 0)

    return s_kernel


def _make_count_kernel():
    W = 16
    mesh = plsc.VectorSubcoreMesh(core_axis_name="c", subcore_axis_name="s")

    @functools.partial(
        pl.kernel,
        out_type=jax.ShapeDtypeStruct((3, NC, NP, W), jnp.float32),
        mesh=mesh,
        compiler_params=pltpu.CompilerParams(use_tc_tiling_on_sc=False),
        scratch_types=[
            pltpu.VMEM((16,), jnp.int32),
            pltpu.VMEM((NCH_C, CH), jnp.int32),
            pltpu.VMEM((CH, W), jnp.float32),
            pltpu.VMEM((CH, W), jnp.float32),
            pltpu.VMEM_SHARED((NP, W), jnp.float32),
        ],
    )
    def count_kernel(dst_hbm, nch_hbm, out_hbm,
                     nch_v, dst_v, ones_v, zrow_v, acc_sh):
        cid = lax.axis_index("c")
        sid = lax.axis_index("s")
        wid = cid * NS + sid

        def _fill(i, _):
            zrow_v[i, pl.ds(0, 16)] = jnp.zeros((16,), jnp.float32)
            ones_v[i, pl.ds(0, 16)] = jnp.ones((16,), jnp.float32)
            return 0
        lax.fori_loop(0, CH, _fill, 0)
        pltpu.sync_copy(nch_hbm, nch_v)
        nv = nch_v[...]

        for p in range(3):
            n_live = nv[p]
            nz = nv[3 + p]
            row0 = sid * nz * CH

            def _zero(z, _):
                pltpu.sync_copy(zrow_v, acc_sh.at[pl.ds(row0 + z * CH, CH)])
                return 0
            lax.fori_loop(0, nz, _zero, 0)
            plsc.subcore_barrier()

            pltpu.sync_copy(dst_hbm.at[wid, p], dst_v)

            def _chunk(j, _):
                pltpu.sync_copy(ones_v, acc_sh.at[dst_v.at[j]], add=True)
                return 0
            lax.fori_loop(0, n_live, _chunk, 0)

            plsc.subcore_barrier()

            def _dump(z, _):
                pltpu.sync_copy(acc_sh.at[pl.ds(row0 + z * CH, CH)],
                                out_hbm.at[p, cid, pl.ds(row0 + z * CH, CH)])
                return 0
            lax.fori_loop(0, nz, _dump, 0)

    return count_kernel


def _lrelu(h):
    return jnp.where(h >= 0, h, 0.01 * h)


def _split_halves(g, out_ref):
    out_ref[0] = g[:, :H]
    out_ref[1] = g[:, H:]


def _make_dis_kernel():
    def body(cnt_ref, dis1_ref, dis2_ref, dq1_ref, dq2_ref):
        row = lax.broadcasted_iota(jnp.int32, (NP, 1), 0)
        for p, n_real, out_ref, dq_ref in (
                (0, N1, dis1_ref, dq1_ref), (1, N2, dis2_ref, dq2_ref)):
            d = cnt_ref[p, 0, :, 0:1] + cnt_ref[p, 1, :, 0:1]
            dis = jnp.where(d > 0, lax.rsqrt(jnp.maximum(d, 1e-30)), 0.0)
            dis = jnp.where(row < n_real, dis, 0.0)
            out_ref[...] = jnp.broadcast_to(dis, (NP, C))
            dq_ref[...] = jnp.reshape(-(dis * dis), (NP,))

    return pl.pallas_call(
        body,
        out_shape=(jax.ShapeDtypeStruct((NP, C), jnp.float32),
                   jax.ShapeDtypeStruct((NP, C), jnp.float32),
                   jax.ShapeDtypeStruct((NP,), jnp.float32),
                   jax.ShapeDtypeStruct((NP,), jnp.float32)),
        compiler_params=pltpu.CompilerParams(
            vmem_limit_bytes=100 * 1024 * 1024))


def _make_scale_kernel():
    def body(h_ref, dis_ref, out_ref):
        _split_halves(h_ref[...] * dis_ref[...], out_ref)

    return pl.pallas_call(
        body, out_shape=jax.ShapeDtypeStruct((NC, NP, H), jnp.float32))


def _make_scale2_kernel():
    def body(s_ref, dis_ref, out_ref):
        d = dis_ref[...]
        _split_halves(-(d * d) * s_ref[...], out_ref)

    return pl.pallas_call(
        body, out_shape=jax.ShapeDtypeStruct((NC, NP, H), jnp.float32))


@functools.lru_cache(maxsize=None)
def _make_layer_kernel(n_real, do_bn, scale_next):
    inv_n = 1.0 / n_real

    def body(tx0_ref, s1_ref, s2_ref, dis_ref, w_ref, b_ref, gam_ref,
             bet_ref, disn_ref, y_ref, gy_ref):
        tx0 = tx0_ref[...]
        d = dis_ref[...]
        tx1 = -d * s1_ref[...]
        tx2 = -2.0 * d * s2_ref[...] - tx0
        pre = (jnp.dot(tx0, w_ref[0], preferred_element_type=jnp.float32)
               + jnp.dot(tx1, w_ref[1], preferred_element_type=jnp.float32)
               + jnp.dot(tx2, w_ref[2], preferred_element_type=jnp.float32)
               + b_ref[...])
        row = lax.broadcasted_iota(jnp.int32, (NP, C), 0)
        pre = jnp.where(row < n_real, pre, 0.0)
        if do_bn:
            m = jnp.sum(pre, axis=0, keepdims=True) * inv_n
            v = jnp.sum(pre * pre, axis=0, keepdims=True) * inv_n - m * m
            y = gam_ref[...] * (pre - m) * lax.rsqrt(v + 1e-5) + bet_ref[...]
            y = _lrelu(y)
        else:
            y = pre
        y = jnp.where(row < n_real, y, 0.0)
        y_ref[...] = y
        if scale_next:
            _split_halves(disn_ref[...] * y, gy_ref)
        else:
            _split_halves(y, gy_ref)

    out_shape = (jax.ShapeDtypeStruct((NP, C), jnp.float32),
                 jax.ShapeDtypeStruct((NC, NP, H), jnp.float32))
    return pl.pallas_call(body, out_shape=out_shape)


@functools.lru_cache(maxsize=None)
def _make_pool_kernel(n_real):
    inv_n = 1.0 / n_real

    def body(ps_ref, cntp_ref, gam_ref, bet_ref, dis_ref, y_ref, gy_ref):
        cnt = cntp_ref[2, 0, :, 0:1] + cntp_ref[2, 1, :, 0:1]
        pooled = ps_ref[...] / jnp.broadcast_to(
            jnp.maximum(cnt, 1.0), (NP, C))
        row = lax.broadcasted_iota(jnp.int32, (NP, C), 0)
        pooled = jnp.where(row < n_real, pooled, 0.0)
        m = jnp.sum(pooled, axis=0, keepdims=True) * inv_n
        v = jnp.sum(pooled * pooled, axis=0, keepdims=True) * inv_n - m * m
        y = gam_ref[...] * (pooled - m) * lax.rsqrt(v + 1e-5) + bet_ref[...]
        y = _lrelu(y)
        row = lax.broadcasted_iota(jnp.int32, (NP, C), 0)
        y = jnp.where(row < n_real, y, 0.0)
        y_ref[...] = y
        _split_halves(dis_ref[...] * y, gy_ref)

    out_shape = (jax.ShapeDtypeStruct((NP, C), jnp.float32),
                 jax.ShapeDtypeStruct((NC, NP, H), jnp.float32))
    return pl.pallas_call(body, out_shape=out_shape)


def _pad_edges(idx, pad_val, ways, nch_max, even=False):
    e = idx.shape[0]
    per = ways * CH
    ep = _ceil_to(e, per)
    live = ep // per
    idx = jnp.concatenate([idx, jnp.full((ways * nch_max * CH - e,),
                                         pad_val, jnp.int32)])
    real = idx[:ep].reshape(live, ways, CH).transpose(1, 0, 2)
    fill = jnp.full((ways, nch_max - live, CH), pad_val, jnp.int32)
    return jnp.concatenate([real, fill], axis=1), live


def kernel(x, edge_index1, edge_index2, pool_idx,
           W1, b1, g1, be1, W2, b2, g2, be2,
           W3, b3, g3, be3, W4, b4, g4, be4, W5, b5, g5, be5):
    xp = jnp.zeros((NP, C), jnp.float32).at[:N1].set(x)

    src1, live1 = _pad_edges(edge_index1[0], N1, NS, NCH_S)
    dst1, _ = _pad_edges(edge_index1[1], N1, NS, NCH_S)
    src2, live2 = _pad_edges(edge_index2[0], N2, NS, NCH_S)
    dst2, _ = _pad_edges(edge_index2[1], N2, NS, NCH_S)
    psrc, livep = _pad_edges(jnp.arange(N1, dtype=jnp.int32), N1, NS, NCH_S)
    pdst, _ = _pad_edges(pool_idx.astype(jnp.int32), N2, NS, NCH_S)

    cdst1, clive1 = _pad_edges(edge_index1[1], N1, NW, NCH_C)
    cdst2, clive2 = _pad_edges(edge_index2[1], N2, NW, NCH_C)
    cpdst, clivep = _pad_edges(pool_idx.astype(jnp.int32), N2, NW, NCH_C)

    NZ1 = NP // (NS * CH)
    NZ2 = _ceil_to(N2, NS * CH) // (NS * CH)

    def nvec(live, nz, g2=0):
        v = [live, nz, g2] + [0] * 13
        return jnp.array(v, jnp.int32)

    sk = _make_s_kernel()
    ck = _make_count_kernel()
    scale = _make_scale_kernel()
    scale2 = _make_scale2_kernel()

    def S(g_halves, src, dst, live, nz, dq, want_g2=0):
        s, g2 = sk(g_halves, src, dst, nvec(live, nz, want_g2), dq)
        return s.reshape(NP, C), g2

    cdst = jnp.stack([cdst1, cdst2, cpdst], axis=1)
    cnts = ck(cdst, jnp.array(
        [clive1, clive2, clivep, NZ1, NZ2, NZ2] + [0] * 10, jnp.int32))
    dis1, dis2, dq1, dq2 = _make_dis_kernel()(cnts)

    def cheb_hops(gh, src, dst, live, nz, dq):
        s1, g2 = S(gh, src, dst, live, nz, dq, want_g2=1)
        s2, _ = S(g2, src, dst, live, nz, dq)
        return s1, s2

    s1, s2 = cheb_hops(scale(xp, dis1), src1, dst1, live1, NZ1, dq1)
    y1, gy1 = _make_layer_kernel(N1, True, True)(
        xp, s1, s2, dis1, W1, b1, g1, be1, dis1)

    s1, s2 = cheb_hops(gy1, src1, dst1, live1, NZ1, dq1)
    h2, h2_halves = _make_layer_kernel(N1, False, False)(
        y1, s1, s2, dis1, W2, b2, g2, be2, dis1)

    ps, _ = S(h2_halves, psrc, pdst, livep, NZ2, dq2)
    y, gy = _make_pool_kernel(N2)(ps, cnts, g2, be2, dis2)

    for W, b, gam, bet in ((W3, b3, g3, be3),
                           (W4, b4, g4, be4),
                           (W5, b5, g5, be5)):
        s1, s2 = cheb_hops(gy, src2, dst2, live2, NZ2, dq2)
        y, gy = _make_layer_kernel(N2, True, True)(
            y, s1, s2, dis2, W, b, gam, bet, dis2)

    return y[:N2]


def _make_split_kernel():
    def body(h_ref, out_ref):
        _split_halves(h_ref[...], out_ref)

    return pl.pallas_call(
        body, out_shape=jax.ShapeDtypeStruct((NC, NP, H), jnp.float32))


def _split2(h):
    return _make_split_kernel()(h)


# speedup vs baseline: 1.0836x; 1.0002x over previous
import functools

import jax
import jax.numpy as jnp
from jax import lax
from jax.experimental import pallas as pl
from jax.experimental.pallas import tpu as pltpu
from jax.experimental.pallas import tpu_sc as plsc

N1 = 10000
N2 = 6000
C = 128
H = 64

NC = 2
NS = 16
NW = NC * NS
CH = 128

NP = 10240
RPT = NP // NS

NCH_S = 160
NCH_C = 79


def _ceil_to(x, m):
    return (x + m - 1) // m * m


def _make_s_kernel():
    mesh = plsc.VectorSubcoreMesh(core_axis_name="c", subcore_axis_name="s")

    @functools.partial(
        pl.kernel,
        out_type=(jax.ShapeDtypeStruct((NP, NC, H), jnp.float32),
                  jax.ShapeDtypeStruct((NC, NP, H), jnp.float32)),
        mesh=mesh,
        compiler_params=pltpu.CompilerParams(use_tc_tiling_on_sc=False),
        scratch_types=[
            pltpu.VMEM((16,), jnp.int32),
            pltpu.VMEM((NCH_S, CH), jnp.int32),
            pltpu.VMEM((NCH_S, CH), jnp.int32),
            pltpu.VMEM((1, CH, H), jnp.float32),
            pltpu.VMEM((1, CH, H), jnp.float32),
            pltpu.VMEM((CH, H), jnp.float32),
            pltpu.VMEM((RPT,), jnp.float32),
            pltpu.VMEM_SHARED((NP, H), jnp.float32),
            pltpu.SemaphoreType.DMA,
        ],
    )
    def s_kernel(g_hbm, src_hbm, dst_hbm, nch_hbm, dq_hbm,
                 out_hbm, g2_hbm,
                 nch_v, src_v, dst_v, bufa_v, bufb_v, zrow_v, dq_v, acc_sh,
                 sga):
        cid = lax.axis_index("c")
        sid = lax.axis_index("s")

        pltpu.sync_copy(nch_hbm, nch_v)
        nv = nch_v[...]
        n_live = nv[0]
        nz = nv[1]
        row0 = sid * nz * CH

        def _zfill(i, _):
            for k in range(H // 16):
                zrow_v[i, pl.ds(k * 16, 16)] = jnp.zeros((16,), jnp.float32)
            return 0
        lax.fori_loop(0, CH, _zfill, 0)

        def _zero(z, _):
            pltpu.sync_copy(zrow_v, acc_sh.at[pl.ds(row0 + z * CH, CH)])
            return 0
        lax.fori_loop(0, nz, _zero, 0)
        plsc.subcore_barrier()

        pltpu.sync_copy(src_hbm.at[sid], src_v)
        pltpu.sync_copy(dst_hbm.at[sid], dst_v)

        rows_v = bufa_v.at[0]

        def _chunk(j, _):
            pltpu.async_copy(
                g_hbm.at[cid].at[src_v.at[j]], rows_v, sga).wait()
            pltpu.sync_copy(rows_v, acc_sh.at[dst_v.at[j]], add=True)
            return 0
        lax.fori_loop(0, n_live, _chunk, 0)

        plsc.subcore_barrier()

        def _dump(z, _):
            pltpu.sync_copy(acc_sh.at[pl.ds(row0 + z * CH, CH)],
                            out_hbm.at[pl.ds(row0 + z * CH, CH), cid])
            return 0
        lax.fori_loop(0, nz, _dump, 0)

        want_g2 = nv[2]

        @pl.when(want_g2 == 1)
        def _scaled_dump():
            pltpu.sync_copy(dq_hbm.at[pl.ds(row0, RPT)], dq_v)
            buf = bufa_v.at[0]
            sbuf = bufb_v.at[0]

            def _sdump(z, _):
                zoff = z * CH
                pltpu.sync_copy(acc_sh.at[pl.ds(row0 + zoff, CH)], buf)
                for g in range(CH // 16):
                    dq16 = dq_v[pl.ds(zoff + g * 16, 16)]
                    for k in range(16):
                        r = g * 16 + k
                        s = dq16[k]
                        for c in range(H // 16):
                            sbuf[r, pl.ds(c * 16, 16)] = (
                                buf[r, pl.ds(c * 16, 16)] * s)
                pltpu.sync_copy(
                    sbuf, g2_hbm.at[cid, pl.ds(row0 + zoff, CH)])
                return 0
            lax.fori_loop(0, nz, _sdump, 0)

    return s_kernel


def _make_count_kernel():
    W = 16
    mesh = plsc.VectorSubcoreMesh(core_axis_name="c", subcore_axis_name="s")

    @functools.partial(
        pl.kernel,
        out_type=jax.ShapeDtypeStruct((3, NC, NP, W), jnp.float32),
        mesh=mesh,
        compiler_params=pltpu.CompilerParams(use_tc_tiling_on_sc=False),
        scratch_types=[
            pltpu.VMEM((16,), jnp.int32),
            pltpu.VMEM((NCH_C, CH), jnp.int32),
            pltpu.VMEM((CH, W), jnp.float32),
            pltpu.VMEM((CH, W), jnp.float32),
            pltpu.VMEM_SHARED((NP, W), jnp.float32),
        ],
    )
    def count_kernel(dst_hbm, nch_hbm, out_hbm,
                     nch_v, dst_v, ones_v, zrow_v, acc_sh):
        cid = lax.axis_index("c")
        sid = lax.axis_index("s")
        wid = cid * NS + sid

        def _fill(i, _):
            zrow_v[i, pl.ds(0, 16)] = jnp.zeros((16,), jnp.float32)
            ones_v[i, pl.ds(0, 16)] = jnp.ones((16,), jnp.float32)
            return 0
        lax.fori_loop(0, CH, _fill, 0)
        pltpu.sync_copy(nch_hbm, nch_v)
        nv = nch_v[...]

        for p in range(3):
            n_live = nv[p]
            nz = nv[3 + p]
            row0 = sid * nz * CH

            def _zero(z, _):
                pltpu.sync_copy(zrow_v, acc_sh.at[pl.ds(row0 + z * CH, CH)])
                return 0
            lax.fori_loop(0, nz, _zero, 0)
            plsc.subcore_barrier()

            pltpu.sync_copy(dst_hbm.at[wid, p], dst_v)

            def _chunk(j, _):
                pltpu.sync_copy(ones_v, acc_sh.at[dst_v.at[j]], add=True)
                return 0
            lax.fori_loop(0, n_live, _chunk, 0)

            plsc.subcore_barrier()

            def _dump(z, _):
                pltpu.sync_copy(acc_sh.at[pl.ds(row0 + z * CH, CH)],
                                out_hbm.at[p, cid, pl.ds(row0 + z * CH, CH)])
                return 0
            lax.fori_loop(0, nz, _dump, 0)

    return count_kernel


def _lrelu(h):
    return jnp.where(h >= 0, h, 0.01 * h)


def _split_halves(g, out_ref):
    out_ref[0] = g[:, :H]
    out_ref[1] = g[:, H:]


def _make_dis_kernel():
    def body(cnt_ref, dis1_ref, dis2_ref, dq1_ref, dq2_ref):
        row = lax.broadcasted_iota(jnp.int32, (NP, 1), 0)
        for p, n_real, out_ref, dq_ref in (
                (0, N1, dis1_ref, dq1_ref), (1, N2, dis2_ref, dq2_ref)):
            d = cnt_ref[p, 0, :, 0:1] + cnt_ref[p, 1, :, 0:1]
            dis = jnp.where(d > 0, lax.rsqrt(jnp.maximum(d, 1e-30)), 0.0)
            dis = jnp.where(row < n_real, dis, 0.0)
            out_ref[...] = jnp.broadcast_to(dis, (NP, C))
            dq_ref[...] = jnp.reshape(-(dis * dis), (NP,))

    return pl.pallas_call(
        body,
        out_shape=(jax.ShapeDtypeStruct((NP, C), jnp.float32),
                   jax.ShapeDtypeStruct((NP, C), jnp.float32),
                   jax.ShapeDtypeStruct((NP,), jnp.float32),
                   jax.ShapeDtypeStruct((NP,), jnp.float32)),
        compiler_params=pltpu.CompilerParams(
            vmem_limit_bytes=100 * 1024 * 1024))


def _make_scale_kernel():
    def body(h_ref, dis_ref, out_ref):
        _split_halves(h_ref[...] * dis_ref[...], out_ref)

    return pl.pallas_call(
        body, out_shape=jax.ShapeDtypeStruct((NC, NP, H), jnp.float32))


@functools.lru_cache(maxsize=None)
def _make_layer_kernel(n_real, do_bn, scale_next):
    inv_n = 1.0 / n_real

    def body(tx0_ref, s1_ref, s2_ref, dis_ref, w_ref, b_ref, gam_ref,
             bet_ref, disn_ref, y_ref, gy_ref):
        tx0 = tx0_ref[...]
        d = dis_ref[...]
        tx1 = -d * s1_ref[...]
        tx2 = -2.0 * d * s2_ref[...] - tx0
        pre = (jnp.dot(tx0, w_ref[0], preferred_element_type=jnp.float32)
               + jnp.dot(tx1, w_ref[1], preferred_element_type=jnp.float32)
               + jnp.dot(tx2, w_ref[2], preferred_element_type=jnp.float32)
               + b_ref[...])
        row = lax.broadcasted_iota(jnp.int32, (NP, C), 0)
        pre = jnp.where(row < n_real, pre, 0.0)
        if do_bn:
            m = jnp.sum(pre, axis=0, keepdims=True) * inv_n
            v = jnp.sum(pre * pre, axis=0, keepdims=True) * inv_n - m * m
            y = gam_ref[...] * (pre - m) * lax.rsqrt(v + 1e-5) + bet_ref[...]
            y = _lrelu(y)
        else:
            y = pre
        y = jnp.where(row < n_real, y, 0.0)
        y_ref[...] = y
        if scale_next:
            _split_halves(disn_ref[...] * y, gy_ref)
        else:
            _split_halves(y, gy_ref)

    out_shape = (jax.ShapeDtypeStruct((NP, C), jnp.float32),
                 jax.ShapeDtypeStruct((NC, NP, H), jnp.float32))
    return pl.pallas_call(body, out_shape=out_shape)


@functools.lru_cache(maxsize=None)
def _make_pool_kernel(n_real):
    inv_n = 1.0 / n_real

    def body(ps_ref, cntp_ref, gam_ref, bet_ref, dis_ref, y_ref, gy_ref):
        cnt = cntp_ref[2, 0, :, 0:1] + cntp_ref[2, 1, :, 0:1]
        pooled = ps_ref[...] / jnp.broadcast_to(
            jnp.maximum(cnt, 1.0), (NP, C))
        row = lax.broadcasted_iota(jnp.int32, (NP, C), 0)
        pooled = jnp.where(row < n_real, pooled, 0.0)
        m = jnp.sum(pooled, axis=0, keepdims=True) * inv_n
        v = jnp.sum(pooled * pooled, axis=0, keepdims=True) * inv_n - m * m
        y = gam_ref[...] * (pooled - m) * lax.rsqrt(v + 1e-5) + bet_ref[...]
        y = _lrelu(y)
        row = lax.broadcasted_iota(jnp.int32, (NP, C), 0)
        y = jnp.where(row < n_real, y, 0.0)
        y_ref[...] = y
        _split_halves(dis_ref[...] * y, gy_ref)

    out_shape = (jax.ShapeDtypeStruct((NP, C), jnp.float32),
                 jax.ShapeDtypeStruct((NC, NP, H), jnp.float32))
    return pl.pallas_call(body, out_shape=out_shape)


def _pad_edges(idx, pad_val, ways, nch_max):
    e = idx.shape[0]
    per = ways * CH
    ep = _ceil_to(e, per)
    live = ep // per
    idx = jnp.concatenate([idx, jnp.full((ways * nch_max * CH - e,),
                                         pad_val, jnp.int32)])
    real = idx[:ep].reshape(live, ways, CH).transpose(1, 0, 2)
    fill = jnp.full((ways, nch_max - live, CH), pad_val, jnp.int32)
    return jnp.concatenate([real, fill], axis=1), live


def kernel(x, edge_index1, edge_index2, pool_idx,
           W1, b1, g1, be1, W2, b2, g2, be2,
           W3, b3, g3, be3, W4, b4, g4, be4, W5, b5, g5, be5):
    xp = jnp.zeros((NP, C), jnp.float32).at[:N1].set(x)

    src1, live1 = _pad_edges(edge_index1[0], N1, NS, NCH_S)
    dst1, _ = _pad_edges(edge_index1[1], N1, NS, NCH_S)
    src2, live2 = _pad_edges(edge_index2[0], N2, NS, NCH_S)
    dst2, _ = _pad_edges(edge_index2[1], N2, NS, NCH_S)
    psrc, livep = _pad_edges(jnp.arange(N1, dtype=jnp.int32), N1, NS, NCH_S)
    pdst, _ = _pad_edges(pool_idx.astype(jnp.int32), N2, NS, NCH_S)

    cdst1, clive1 = _pad_edges(edge_index1[1], N1, NW, NCH_C)
    cdst2, clive2 = _pad_edges(edge_index2[1], N2, NW, NCH_C)
    cpdst, clivep = _pad_edges(pool_idx.astype(jnp.int32), N2, NW, NCH_C)

    NZ1 = NP // (NS * CH)
    NZ2 = _ceil_to(N2, NS * CH) // (NS * CH)

    def nvec(live, nz, g2=0):
        v = [live, nz, g2] + [0] * 13
        return jnp.array(v, jnp.int32)

    sk = _make_s_kernel()
    ck = _make_count_kernel()
    scale = _make_scale_kernel()

    def S(g_halves, src, dst, live, nz, dq, want_g2=0):
        s, g2 = sk(g_halves, src, dst, nvec(live, nz, want_g2), dq)
        return s.reshape(NP, C), g2

    cdst = jnp.stack([cdst1, cdst2, cpdst], axis=1)
    cnts = ck(cdst, jnp.array(
        [clive1, clive2, clivep, NZ1, NZ2, NZ2] + [0] * 10, jnp.int32))
    dis1, dis2, dq1, dq2 = _make_dis_kernel()(cnts)

    def cheb_hops(gh, src, dst, live, nz, dq):
        s1, g2 = S(gh, src, dst, live, nz, dq, want_g2=1)
        s2, _ = S(g2, src, dst, live, nz, dq)
        return s1, s2

    s1, s2 = cheb_hops(scale(xp, dis1), src1, dst1, live1, NZ1, dq1)
    y1, gy1 = _make_layer_kernel(N1, True, True)(
        xp, s1, s2, dis1, W1, b1, g1, be1, dis1)

    s1, s2 = cheb_hops(gy1, src1, dst1, live1, NZ1, dq1)
    h2, h2_halves = _make_layer_kernel(N1, False, False)(
        y1, s1, s2, dis1, W2, b2, g2, be2, dis1)

    ps, _ = S(h2_halves, psrc, pdst, livep, NZ2, dq2)
    y, gy = _make_pool_kernel(N2)(ps, cnts, g2, be2, dis2)

    for W, b, gam, bet in ((W3, b3, g3, be3),
                           (W4, b4, g4, be4),
                           (W5, b5, g5, be5)):
        s1, s2 = cheb_hops(gy, src2, dst2, live2, NZ2, dq2)
        y, gy = _make_layer_kernel(N2, True, True)(
            y, s1, s2, dis2, W, b, gam, bet, dis2)

    return y[:N2]


# speedup vs baseline: 1.1297x; 1.0425x over previous
import functools

import jax
import jax.numpy as jnp
from jax import lax
from jax.experimental import pallas as pl
from jax.experimental.pallas import tpu as pltpu
from jax.experimental.pallas import tpu_sc as plsc

N1 = 10000
N2 = 6000
C = 128
H = 64

NC = 2
NS = 16
NW = NC * NS
CH = 128

NP = 10240
RPT = NP // NS

NCH_S = 160
NCH_C = 79


def _ceil_to(x, m):
    return (x + m - 1) // m * m


def _make_s_kernel():
    mesh = plsc.VectorSubcoreMesh(core_axis_name="c", subcore_axis_name="s")

    @functools.partial(
        pl.kernel,
        out_type=(jax.ShapeDtypeStruct((NP, NC, H), jnp.float32),
                  jax.ShapeDtypeStruct((NC, NP, H), jnp.float32)),
        mesh=mesh,
        compiler_params=pltpu.CompilerParams(use_tc_tiling_on_sc=False),
        scratch_types=[
            pltpu.VMEM((16,), jnp.int32),
            pltpu.VMEM((NCH_S, CH), jnp.int32),
            pltpu.VMEM((NCH_S, CH), jnp.int32),
            pltpu.VMEM((1, CH, H), jnp.float32),
            pltpu.VMEM((1, CH, H), jnp.float32),
            pltpu.VMEM((CH, H), jnp.float32),
            pltpu.VMEM((RPT,), jnp.float32),
            pltpu.VMEM_SHARED((NP, H), jnp.float32),
            pltpu.SemaphoreType.DMA,
        ],
    )
    def s_kernel(g_hbm, src_hbm, dst_hbm, nch_hbm, dq_hbm,
                 out_hbm, g2_hbm,
                 nch_v, src_v, dst_v, bufa_v, bufb_v, zrow_v, dq_v, acc_sh,
                 sga):
        cid = lax.axis_index("c")
        sid = lax.axis_index("s")

        pltpu.sync_copy(nch_hbm, nch_v)
        nv = nch_v[...]
        n_live = nv[0]
        nz = nv[1]
        row0 = sid * nz * CH

        def _zfill(i, _):
            for k in range(H // 16):
                zrow_v[i, pl.ds(k * 16, 16)] = jnp.zeros((16,), jnp.float32)
            return 0
        lax.fori_loop(0, CH, _zfill, 0)

        def _zero(z, _):
            pltpu.sync_copy(zrow_v, acc_sh.at[pl.ds(row0 + z * CH, CH)])
            return 0
        lax.fori_loop(0, nz, _zero, 0)
        plsc.subcore_barrier()

        pltpu.sync_copy(src_hbm.at[sid], src_v)
        pltpu.sync_copy(dst_hbm.at[sid], dst_v)

        rows_v = bufa_v.at[0]

        def _chunk(j, _):
            pltpu.async_copy(
                g_hbm.at[cid].at[src_v.at[j]], rows_v, sga).wait()
            pltpu.sync_copy(rows_v, acc_sh.at[dst_v.at[j]], add=True)
            return 0
        lax.fori_loop(0, n_live, _chunk, 0)

        plsc.subcore_barrier()

        def _dump(z, _):
            pltpu.sync_copy(acc_sh.at[pl.ds(row0 + z * CH, CH)],
                            out_hbm.at[pl.ds(row0 + z * CH, CH), cid])
            return 0
        lax.fori_loop(0, nz, _dump, 0)

        want_g2 = nv[2]

        @pl.when(want_g2 == 1)
        def _scaled_dump():
            pltpu.sync_copy(dq_hbm.at[pl.ds(row0, RPT)], dq_v)
            buf = bufa_v.at[0]
            sbuf = bufb_v.at[0]

            def _sdump(z, _):
                zoff = z * CH
                pltpu.sync_copy(acc_sh.at[pl.ds(row0 + zoff, CH)], buf)
                for g in range(CH // 16):
                    dq16 = dq_v[pl.ds(zoff + g * 16, 16)]
                    for k in range(16):
                        r = g * 16 + k
                        s = dq16[k]
                        for c in range(H // 16):
                            sbuf[r, pl.ds(c * 16, 16)] = (
                                buf[r, pl.ds(c * 16, 16)] * s)
                pltpu.sync_copy(
                    sbuf, g2_hbm.at[cid, pl.ds(row0 + zoff, CH)])
                return 0
            lax.fori_loop(0, nz, _sdump, 0)

    return s_kernel


NCH2 = 94
NR2 = 6144
RPT2 = NR2 // NS


def _make_s2_kernel():
    mesh = plsc.VectorSubcoreMesh(core_axis_name="c", subcore_axis_name="s")

    @functools.partial(
        pl.kernel,
        out_type=(jax.ShapeDtypeStruct((NP, NC, H), jnp.float32),
                  jax.ShapeDtypeStruct((NC, NP, H), jnp.float32)),
        mesh=mesh,
        compiler_params=pltpu.CompilerParams(use_tc_tiling_on_sc=False),
        scratch_types=[
            pltpu.VMEM((16,), jnp.int32),
            pltpu.VMEM((NCH2, CH), jnp.int32),
            pltpu.VMEM((NCH2, CH), jnp.int32),
            pltpu.VMEM((CH, H), jnp.float32),
            pltpu.VMEM((CH, H), jnp.float32),
            pltpu.VMEM((CH, H), jnp.float32),
            pltpu.VMEM((RPT2,), jnp.float32),
            pltpu.VMEM_SHARED((NR2, H), jnp.float32),
            pltpu.VMEM_SHARED((NR2, H), jnp.float32),
            pltpu.SemaphoreType.DMA,
        ],
    )
    def s2_kernel(g_hbm, src_hbm, dst_hbm, nch_hbm, dq_hbm,
                  out_hbm, g2_hbm,
                  nch_v, src_v, dst_v, rows_v, sbuf_v, zrow_v, dq_v,
                  acc_sh, tab_sh, sga):
        cid = lax.axis_index("c")
        sid = lax.axis_index("s")
        row0 = sid * RPT2

        pltpu.sync_copy(nch_hbm, nch_v)
        nv = nch_v[...]

        def _zfill(i, _):
            for k in range(H // 16):
                zrow_v[i, pl.ds(k * 16, 16)] = jnp.zeros((16,), jnp.float32)
            return 0
        lax.fori_loop(0, CH, _zfill, 0)

        pltpu.sync_copy(g_hbm.at[cid, pl.ds(row0, RPT2)],
                        tab_sh.at[pl.ds(row0, RPT2)])
        for z in range(RPT2 // CH):
            pltpu.sync_copy(zrow_v, acc_sh.at[pl.ds(row0 + z * CH, CH)])
        plsc.subcore_barrier()

        pltpu.sync_copy(src_hbm.at[sid, pl.ds(0, NCH2)], src_v)
        pltpu.sync_copy(dst_hbm.at[sid, pl.ds(0, NCH2)], dst_v)

        def _chunk(j, _):
            pltpu.async_copy(
                tab_sh.at[src_v.at[j]], rows_v, sga).wait()
            pltpu.sync_copy(rows_v, acc_sh.at[dst_v.at[j]], add=True)
            return 0
        lax.fori_loop(0, NCH2, _chunk, 0)

        plsc.subcore_barrier()

        for z in range(RPT2 // CH):
            pltpu.sync_copy(acc_sh.at[pl.ds(row0 + z * CH, CH)],
                            out_hbm.at[pl.ds(row0 + z * CH, CH), cid])

        want_g2 = nv[2]

        @pl.when(want_g2 == 1)
        def _scaled_dump():
            pltpu.sync_copy(dq_hbm.at[pl.ds(row0, RPT2)], dq_v)

            def _sdump(z, _):
                zoff = z * CH
                pltpu.sync_copy(acc_sh.at[pl.ds(row0 + zoff, CH)], rows_v)
                for g in range(CH // 16):
                    dq16 = dq_v[pl.ds(zoff + g * 16, 16)]
                    for k in range(16):
                        r = g * 16 + k
                        s = dq16[k]
                        for c in range(H // 16):
                            sbuf_v[r, pl.ds(c * 16, 16)] = (
                                rows_v[r, pl.ds(c * 16, 16)] * s)
                pltpu.sync_copy(
                    sbuf_v, g2_hbm.at[cid, pl.ds(row0 + zoff, CH)])
                return 0
            lax.fori_loop(0, RPT2 // CH, _sdump, 0)

    return s2_kernel


def _make_count_kernel():
    W = 16
    mesh = plsc.VectorSubcoreMesh(core_axis_name="c", subcore_axis_name="s")

    @functools.partial(
        pl.kernel,
        out_type=jax.ShapeDtypeStruct((3, NC, NP, W), jnp.float32),
        mesh=mesh,
        compiler_params=pltpu.CompilerParams(use_tc_tiling_on_sc=False),
        scratch_types=[
            pltpu.VMEM((16,), jnp.int32),
            pltpu.VMEM((NCH_C, CH), jnp.int32),
            pltpu.VMEM((CH, W), jnp.float32),
            pltpu.VMEM((CH, W), jnp.float32),
            pltpu.VMEM_SHARED((NP, W), jnp.float32),
        ],
    )
    def count_kernel(dst_hbm, nch_hbm, out_hbm,
                     nch_v, dst_v, ones_v, zrow_v, acc_sh):
        cid = lax.axis_index("c")
        sid = lax.axis_index("s")
        wid = cid * NS + sid

        def _fill(i, _):
            zrow_v[i, pl.ds(0, 16)] = jnp.zeros((16,), jnp.float32)
            ones_v[i, pl.ds(0, 16)] = jnp.ones((16,), jnp.float32)
            return 0
        lax.fori_loop(0, CH, _fill, 0)
        pltpu.sync_copy(nch_hbm, nch_v)
        nv = nch_v[...]

        for p in range(3):
            n_live = nv[p]
            nz = nv[3 + p]
            row0 = sid * nz * CH

            def _zero(z, _):
                pltpu.sync_copy(zrow_v, acc_sh.at[pl.ds(row0 + z * CH, CH)])
                return 0
            lax.fori_loop(0, nz, _zero, 0)
            plsc.subcore_barrier()

            pltpu.sync_copy(dst_hbm.at[wid, p], dst_v)

            def _chunk(j, _):
                pltpu.sync_copy(ones_v, acc_sh.at[dst_v.at[j]], add=True)
                return 0
            lax.fori_loop(0, n_live, _chunk, 0)

            plsc.subcore_barrier()

            def _dump(z, _):
                pltpu.sync_copy(acc_sh.at[pl.ds(row0 + z * CH, CH)],
                                out_hbm.at[p, cid, pl.ds(row0 + z * CH, CH)])
                return 0
            lax.fori_loop(0, nz, _dump, 0)

    return count_kernel


def _lrelu(h):
    return jnp.where(h >= 0, h, 0.01 * h)


def _split_halves(g, out_ref):
    out_ref[0] = g[:, :H]
    out_ref[1] = g[:, H:]


def _make_dis_kernel():
    def body(cnt_ref, dis1_ref, dis2_ref, dq1_ref, dq2_ref):
        row = lax.broadcasted_iota(jnp.int32, (NP, 1), 0)
        for p, n_real, out_ref, dq_ref in (
                (0, N1, dis1_ref, dq1_ref), (1, N2, dis2_ref, dq2_ref)):
            d = cnt_ref[p, 0, :, 0:1] + cnt_ref[p, 1, :, 0:1]
            dis = jnp.where(d > 0, lax.rsqrt(jnp.maximum(d, 1e-30)), 0.0)
            dis = jnp.where(row < n_real, dis, 0.0)
            out_ref[...] = jnp.broadcast_to(dis, (NP, C))
            dq_ref[...] = jnp.reshape(-(dis * dis), (NP,))

    return pl.pallas_call(
        body,
        out_shape=(jax.ShapeDtypeStruct((NP, C), jnp.float32),
                   jax.ShapeDtypeStruct((NP, C), jnp.float32),
                   jax.ShapeDtypeStruct((NP,), jnp.float32),
                   jax.ShapeDtypeStruct((NP,), jnp.float32)),
        compiler_params=pltpu.CompilerParams(
            vmem_limit_bytes=100 * 1024 * 1024))


def _make_scale_kernel():
    def body(h_ref, dis_ref, out_ref):
        _split_halves(h_ref[...] * dis_ref[...], out_ref)

    return pl.pallas_call(
        body, out_shape=jax.ShapeDtypeStruct((NC, NP, H), jnp.float32))


@functools.lru_cache(maxsize=None)
def _make_layer_kernel(n_real, do_bn, scale_next):
    inv_n = 1.0 / n_real

    def body(tx0_ref, s1_ref, s2_ref, dis_ref, w_ref, b_ref, gam_ref,
             bet_ref, disn_ref, y_ref, gy_ref):
        tx0 = tx0_ref[...]
        d = dis_ref[...]
        tx1 = -d * s1_ref[...]
        tx2 = -2.0 * d * s2_ref[...] - tx0
        pre = (jnp.dot(tx0, w_ref[0], preferred_element_type=jnp.float32)
               + jnp.dot(tx1, w_ref[1], preferred_element_type=jnp.float32)
               + jnp.dot(tx2, w_ref[2], preferred_element_type=jnp.float32)
               + b_ref[...])
        row = lax.broadcasted_iota(jnp.int32, (NP, C), 0)
        pre = jnp.where(row < n_real, pre, 0.0)
        if do_bn:
            m = jnp.sum(pre, axis=0, keepdims=True) * inv_n
            v = jnp.sum(pre * pre, axis=0, keepdims=True) * inv_n - m * m
            y = gam_ref[...] * (pre - m) * lax.rsqrt(v + 1e-5) + bet_ref[...]
            y = _lrelu(y)
        else:
            y = pre
        y = jnp.where(row < n_real, y, 0.0)
        y_ref[...] = y
        if scale_next:
            _split_halves(disn_ref[...] * y, gy_ref)
        else:
            _split_halves(y, gy_ref)

    out_shape = (jax.ShapeDtypeStruct((NP, C), jnp.float32),
                 jax.ShapeDtypeStruct((NC, NP, H), jnp.float32))
    return pl.pallas_call(body, out_shape=out_shape)


@functools.lru_cache(maxsize=None)
def _make_pool_kernel(n_real):
    inv_n = 1.0 / n_real

    def body(ps_ref, cntp_ref, gam_ref, bet_ref, dis_ref, y_ref, gy_ref):
        cnt = cntp_ref[2, 0, :, 0:1] + cntp_ref[2, 1, :, 0:1]
        pooled = ps_ref[...] / jnp.broadcast_to(
            jnp.maximum(cnt, 1.0), (NP, C))
        row = lax.broadcasted_iota(jnp.int32, (NP, C), 0)
        pooled = jnp.where(row < n_real, pooled, 0.0)
        m = jnp.sum(pooled, axis=0, keepdims=True) * inv_n
        v = jnp.sum(pooled * pooled, axis=0, keepdims=True) * inv_n - m * m
        y = gam_ref[...] * (pooled - m) * lax.rsqrt(v + 1e-5) + bet_ref[...]
        y = _lrelu(y)
        row = lax.broadcasted_iota(jnp.int32, (NP, C), 0)
        y = jnp.where(row < n_real, y, 0.0)
        y_ref[...] = y
        _split_halves(dis_ref[...] * y, gy_ref)

    out_shape = (jax.ShapeDtypeStruct((NP, C), jnp.float32),
                 jax.ShapeDtypeStruct((NC, NP, H), jnp.float32))
    return pl.pallas_call(body, out_shape=out_shape)


def _pad_edges(idx, pad_val, ways, nch_max):
    e = idx.shape[0]
    per = ways * CH
    ep = _ceil_to(e, per)
    live = ep // per
    idx = jnp.concatenate([idx, jnp.full((ways * nch_max * CH - e,),
                                         pad_val, jnp.int32)])
    real = idx[:ep].reshape(live, ways, CH).transpose(1, 0, 2)
    fill = jnp.full((ways, nch_max - live, CH), pad_val, jnp.int32)
    return jnp.concatenate([real, fill], axis=1), live


def kernel(x, edge_index1, edge_index2, pool_idx,
           W1, b1, g1, be1, W2, b2, g2, be2,
           W3, b3, g3, be3, W4, b4, g4, be4, W5, b5, g5, be5):
    xp = jnp.zeros((NP, C), jnp.float32).at[:N1].set(x)

    src1, live1 = _pad_edges(edge_index1[0], N1, NS, NCH_S)
    dst1, _ = _pad_edges(edge_index1[1], N1, NS, NCH_S)
    src2, live2 = _pad_edges(edge_index2[0], N2, NS, NCH_S)
    dst2, _ = _pad_edges(edge_index2[1], N2, NS, NCH_S)
    psrc, livep = _pad_edges(jnp.arange(N1, dtype=jnp.int32), N1, NS, NCH_S)
    pdst, _ = _pad_edges(pool_idx.astype(jnp.int32), N2, NS, NCH_S)

    cdst1, clive1 = _pad_edges(edge_index1[1], N1, NW, NCH_C)
    cdst2, clive2 = _pad_edges(edge_index2[1], N2, NW, NCH_C)
    cpdst, clivep = _pad_edges(pool_idx.astype(jnp.int32), N2, NW, NCH_C)

    NZ1 = NP // (NS * CH)
    NZ2 = _ceil_to(N2, NS * CH) // (NS * CH)

    def nvec(live, nz, g2=0):
        v = [live, nz, g2] + [0] * 13
        return jnp.array(v, jnp.int32)

    sk = _make_s_kernel()
    sk2 = _make_s2_kernel()
    ck = _make_count_kernel()

    def S(g_halves, src, dst, live, nz, dq, want_g2=0):
        s, g2 = sk(g_halves, src, dst, nvec(live, nz, want_g2), dq)
        return s.reshape(NP, C), g2

    cdst = jnp.stack([cdst1, cdst2, cpdst], axis=1)
    cnts = ck(cdst, jnp.array(
        [clive1, clive2, clivep, NZ1, NZ2, NZ2] + [0] * 10, jnp.int32))
    dis1, dis2, dq1, dq2 = _make_dis_kernel()(cnts)

    def cheb_hops(gh, src, dst, live, nz, dq):
        s1, g2 = S(gh, src, dst, live, nz, dq, want_g2=1)
        s2, _ = S(g2, src, dst, live, nz, dq)
        return s1, s2

    def cheb_hops2(gh, dq):
        s1, g2 = sk2(gh, src2, dst2, nvec(0, 0, 1), dq)
        s2, _ = sk2(g2, src2, dst2, nvec(0, 0, 0), dq)
        return s1.reshape(NP, C), s2.reshape(NP, C)

    g0 = _make_scale_kernel()(xp, dis1)
    s1, s2 = cheb_hops(g0, src1, dst1, live1, NZ1, dq1)
    y1, gy1 = _make_layer_kernel(N1, True, True)(
        xp, s1, s2, dis1, W1, b1, g1, be1, dis1)

    s1, s2 = cheb_hops(gy1, src1, dst1, live1, NZ1, dq1)
    h2, h2_halves = _make_layer_kernel(N1, False, False)(
        y1, s1, s2, dis1, W2, b2, g2, be2, dis1)

    ps, _ = S(h2_halves, psrc, pdst, livep, NZ2, dq2)
    y, gy = _make_pool_kernel(N2)(ps, cnts, g2, be2, dis2)

    for W, b, gam, bet in ((W3, b3, g3, be3),
                           (W4, b4, g4, be4),
                           (W5, b5, g5, be5)):
        s1, s2 = cheb_hops2(gy, dq2)
        y, gy = _make_layer_kernel(N2, True, True)(
            y, s1, s2, dis2, W, b, gam, bet, dis2)

    return y[:N2]
